# baseline (speedup 1.0000x reference)
import jax
import jax.numpy as jnp
from jax.experimental import pallas as pl
from jax.experimental.pallas import tpu as pltpu


_BM = 512
_BK = 512


def _fused_kernel(
    v_ref,
    adj_ref,
    w1_ref,
    w2_ref,
    wout_ref,
    bout_ref,
    out_ref,
    adjb_ref,
    z1_ref,
    z2_ref,
    acc1_ref,
    acc2_ref,
):
    p = pl.program_id(0)
    i = pl.program_id(1)
    k = pl.program_id(2)
    nk = pl.num_programs(2)

    @pl.when(p == 0)
    def _():
        @pl.when(i == 0)
        def _():
            z1_ref[pl.ds(k * _BK, _BK), :] = jnp.dot(
                v_ref[...].astype(jnp.bfloat16),
                w1_ref[...].astype(jnp.bfloat16),
                preferred_element_type=jnp.float32,
            ).astype(jnp.bfloat16)

        a = adj_ref[...].astype(jnp.bfloat16)
        adjb_ref[pl.ds(i * _BM, _BM), pl.ds(k * _BK, _BK)] = a
        part = jnp.dot(
            a,
            z1_ref[pl.ds(k * _BK, _BK), :],
            preferred_element_type=jnp.float32,
        )

        @pl.when(k == 0)
        def _():
            acc1_ref[...] = part

        @pl.when(k > 0)
        def _():
            acc1_ref[...] += part

        @pl.when(k == nk - 1)
        def _():
            h = jnp.maximum(acc1_ref[...], 0.0).astype(jnp.bfloat16)
            z2_ref[pl.ds(i * _BM, _BM), :] = jnp.dot(
                h,
                w2_ref[...].astype(jnp.bfloat16),
                preferred_element_type=jnp.float32,
            ).astype(jnp.bfloat16)

    @pl.when(p == 1)
    def _():
        part = jnp.dot(
            adjb_ref[pl.ds(i * _BM, _BM), pl.ds(k * _BK, _BK)],
            z2_ref[pl.ds(k * _BK, _BK), :],
            preferred_element_type=jnp.float32,
        )

        @pl.when(k == 0)
        def _():
            acc2_ref[...] = part

        @pl.when(k > 0)
        def _():
            acc2_ref[...] += part

        @pl.when(k == nk - 1)
        def _():
            h = jnp.maximum(acc2_ref[...], 0.0)
            x = jnp.sum(h, axis=1)
            contrib = jnp.sum(wout_ref[...] * x[None, :], axis=1)

            @pl.when(i == 0)
            def _():
                out_ref[...] = bout_ref[...]

            out_ref[...] += contrib[None, :]


def kernel(v, adj, W1, W2, W_out, b_out):
    N, F_IN = v.shape
    H1 = W1.shape[1]
    H2 = W2.shape[1]
    LABEL = W_out.shape[0]
    nt = N // _BM
    nk = N // _BK

    out2d = pl.pallas_call(
        _fused_kernel,
        grid=(2, nt, nk),
        in_specs=[
            pl.BlockSpec(
                (_BK, F_IN),
                lambda p, i, k: (jnp.where((p == 0) & (i == 0), k, nk - 1), 0),
            ),
            pl.BlockSpec(
                (_BM, _BK),
                lambda p, i, k: (
                    jnp.where(p == 0, i, nt - 1),
                    jnp.where(p == 0, k, nk - 1),
                ),
            ),
            pl.BlockSpec((F_IN, H1), lambda p, i, k: (0, 0)),
            pl.BlockSpec((H1, H2), lambda p, i, k: (0, 0)),
            pl.BlockSpec((LABEL, _BM), lambda p, i, k: (0, i)),
            pl.BlockSpec((1, LABEL), lambda p, i, k: (0, 0)),
        ],
        out_specs=pl.BlockSpec((1, LABEL), lambda p, i, k: (0, 0)),
        out_shape=jax.ShapeDtypeStruct((1, LABEL), jnp.float32),
        scratch_shapes=[
            pltpu.VMEM((N, N), jnp.bfloat16),
            pltpu.VMEM((N, H1), jnp.bfloat16),
            pltpu.VMEM((N, H2), jnp.bfloat16),
            pltpu.VMEM((_BM, H1), jnp.float32),
            pltpu.VMEM((_BM, H2), jnp.float32),
        ],
        compiler_params=pltpu.CompilerParams(
            vmem_limit_bytes=128 * 1024 * 1024,
        ),
    )(v, adj, W1, W2, W_out, b_out.reshape(1, LABEL))

    return out2d.reshape(LABEL)


# speedup vs baseline: 2.0459x; 2.0459x over previous
import jax
import jax.numpy as jnp
from jax.experimental import pallas as pl
from jax.experimental.pallas import tpu as pltpu


_RB = 256
_VB = 512


def _kern(
    v_hbm,
    adj_hbm,
    w1_ref,
    w2_ref,
    wout_ref,
    bout_ref,
    out_ref,
    adjb_ref,
    z1_ref,
    z2_ref,
    vbuf_ref,
    abuf_ref,
    vsem,
    asem,
):
    N = adj_hbm.shape[0]
    nv = N // _VB
    nb = N // _RB

    w1b = w1_ref[...].astype(jnp.bfloat16)
    w2b = w2_ref[...].astype(jnp.bfloat16)

    pltpu.make_async_copy(
        adj_hbm.at[pl.ds(0, _RB), :], abuf_ref.at[0], asem.at[0]
    ).start()
    pltpu.make_async_copy(
        v_hbm.at[pl.ds(0, _VB), :], vbuf_ref.at[0], vsem.at[0]
    ).start()
    for t in range(nv):
        if t + 1 < nv:
            pltpu.make_async_copy(
                v_hbm.at[pl.ds((t + 1) * _VB, _VB), :],
                vbuf_ref.at[(t + 1) % 2],
                vsem.at[(t + 1) % 2],
            ).start()
        pltpu.make_async_copy(
            v_hbm.at[pl.ds(t * _VB, _VB), :], vbuf_ref.at[t % 2], vsem.at[t % 2]
        ).wait()
        z1_ref[pl.ds(t * _VB, _VB), :] = jnp.dot(
            vbuf_ref[t % 2].astype(jnp.bfloat16),
            w1b,
            preferred_element_type=jnp.float32,
        ).astype(jnp.bfloat16)

    pltpu.make_async_copy(
        adj_hbm.at[pl.ds(_RB, _RB), :], abuf_ref.at[1], asem.at[1]
    ).start()

    def _l1_dot(b):
        h = jnp.dot(
            adjb_ref[pl.ds(b * _RB, _RB), :],
            z1_ref[...],
            preferred_element_type=jnp.float32,
        )
        h = jnp.maximum(h, 0.0).astype(jnp.bfloat16)
        z2_ref[pl.ds(b * _RB, _RB), :] = jnp.dot(
            h, w2b, preferred_element_type=jnp.float32
        ).astype(jnp.bfloat16)

    for t in range(nb):
        pltpu.make_async_copy(
            adj_hbm.at[pl.ds(t * _RB, _RB), :], abuf_ref.at[t % 2], asem.at[t % 2]
        ).wait()
        adjb_ref[pl.ds(t * _RB, _RB), :] = abuf_ref[t % 2].astype(jnp.bfloat16)
        if t + 2 < nb:
            pltpu.make_async_copy(
                adj_hbm.at[pl.ds((t + 2) * _RB, _RB), :],
                abuf_ref.at[t % 2],
                asem.at[t % 2],
            ).start()
        if t >= 1:
            _l1_dot(t - 1)
    _l1_dot(nb - 1)

    out_ref[...] = bout_ref[...]
    for t in range(nb):
        h = jnp.dot(
            adjb_ref[pl.ds(t * _RB, _RB), :],
            z2_ref[...],
            preferred_element_type=jnp.float32,
        )
        h = jnp.maximum(h, 0.0)
        x = jnp.sum(h, axis=1)
        contrib = jnp.sum(
            wout_ref[:, pl.ds(t * _RB, _RB)] * x[None, :], axis=1
        )
        out_ref[...] += contrib[None, :]


def kernel(v, adj, W1, W2, W_out, b_out):
    N, F_IN = v.shape
    H1 = W1.shape[1]
    H2 = W2.shape[1]
    LABEL = W_out.shape[0]

    out2d = pl.pallas_call(
        _kern,
        in_specs=[
            pl.BlockSpec(memory_space=pltpu.MemorySpace.HBM),
            pl.BlockSpec(memory_space=pltpu.MemorySpace.HBM),
            pl.BlockSpec(memory_space=pltpu.MemorySpace.VMEM),
            pl.BlockSpec(memory_space=pltpu.MemorySpace.VMEM),
            pl.BlockSpec(memory_space=pltpu.MemorySpace.VMEM),
            pl.BlockSpec(memory_space=pltpu.MemorySpace.VMEM),
        ],
        out_specs=pl.BlockSpec(memory_space=pltpu.MemorySpace.VMEM),
        out_shape=jax.ShapeDtypeStruct((1, LABEL), jnp.float32),
        scratch_shapes=[
            pltpu.VMEM((N, N), jnp.bfloat16),
            pltpu.VMEM((N, H1), jnp.bfloat16),
            pltpu.VMEM((N, H2), jnp.bfloat16),
            pltpu.VMEM((2, _VB, F_IN), jnp.float32),
            pltpu.VMEM((2, _RB, N), jnp.float32),
            pltpu.SemaphoreType.DMA((2,)),
            pltpu.SemaphoreType.DMA((2,)),
        ],
        compiler_params=pltpu.CompilerParams(
            vmem_limit_bytes=128 * 1024 * 1024,
        ),
    )(v, adj, W1, W2, W_out, b_out.reshape(1, LABEL))

    return out2d.reshape(LABEL)
